# baseline (speedup 1.0000x reference)
import functools

import jax
import jax.numpy as jnp
from jax import lax
from jax.experimental import pallas as pl
from jax.experimental.pallas import tpu as pltpu
from jax.experimental.pallas import tpu_sc as plsc

D = 768
DP = D // 2
V = 8192
B = 4 * 8192
NW = 32
RPW = B // NW
C = 8
NCHUNK = RPW // C
NB = 8
LANES = 16

_mesh = plsc.VectorSubcoreMesh(core_axis_name="c", subcore_axis_name="s")


@functools.partial(
    pl.kernel,
    mesh=_mesh,
    out_type=jax.ShapeDtypeStruct((B, D), jnp.float32),
    scratch_types=[
        pltpu.VMEM((NCHUNK, C), jnp.int32),
        pltpu.VMEM((NB, C, D), jnp.float32),
        pltpu.VMEM((NB, C, DP), jnp.int32),
    ] + [pltpu.SemaphoreType.DMA] * (3 * NB),
)
def _prepare_encoder(src_hbm, pos_hbm, tab_hbm, out_hbm,
                     idx_v, src_v, gath_v, *sems):
    sem_s = sems[0:NB]
    sem_g = sems[NB:2 * NB]
    sem_o = sems[2 * NB:3 * NB]
    wid = lax.axis_index("s") * 2 + lax.axis_index("c")
    base = wid * RPW
    pltpu.sync_copy(pos_hbm.at[wid], idx_v)

    def start_in(g, b):
        rb = base + g * C
        pltpu.async_copy(src_hbm.at[pl.ds(rb, C)], src_v.at[b], sem_s[b])
        pltpu.async_copy(tab_hbm.at[idx_v.at[g]], gath_v.at[b], sem_g[b])

    def wait_in(g, b):
        rb = base + g * C
        pltpu.make_async_copy(src_hbm.at[pl.ds(rb, C)], src_v.at[b],
                              sem_s[b]).wait()
        pltpu.make_async_copy(tab_hbm.at[idx_v.at[g]], gath_v.at[b],
                              sem_g[b]).wait()

    def start_out(g, b):
        rb = base + g * C
        pltpu.async_copy(src_v.at[b], out_hbm.at[pl.ds(rb, C)], sem_o[b])

    def wait_out(g, b):
        rb = base + g * C
        pltpu.make_async_copy(src_v.at[b], out_hbm.at[pl.ds(rb, C)],
                              sem_o[b]).wait()

    def compute(b):
        def row(r, cc):
            for c in range(DP // LANES):
                x = gath_v[b, r, pl.ds(c * LANES, LANES)]
                lo = jax.lax.bitcast_convert_type(x << 16, jnp.float32)
                hi = jax.lax.bitcast_convert_type(x & jnp.int32(-65536), jnp.float32)
                plsc.addupdate(src_v.at[b, r, pl.ds(c * 2 * LANES, LANES)],
                               lo)
                plsc.addupdate(
                    src_v.at[b, r, pl.ds(c * 2 * LANES + LANES, LANES)], hi)
            return cc
        lax.fori_loop(0, C, row, 0)

    for g0 in range(NB):
        start_in(g0, g0)

    def block(i, carry):
        for bb in range(NB):
            g = NB * i + bb
            b2 = (bb + 4) % NB

            @pl.when(jnp.logical_and(g >= 4, g + 4 < NCHUNK))
            def _():
                wait_out(g - 4, b2)
                start_in(g + 4, b2)

            wait_in(g, bb)
            compute(bb)
            start_out(g, bb)
        return carry

    lax.fori_loop(0, NCHUNK // NB, block, 0)
    for k in range(NB):
        g = NCHUNK - NB + k
        wait_out(g, g % NB)


def kernel(src_word, src_pos, emb_table):
    src_flat = src_word.reshape(B, D)
    pos = src_pos.reshape(NW, NCHUNK, C)
    tab16 = jax.lax.bitcast_convert_type(
        emb_table.astype(jnp.bfloat16), jnp.uint16).astype(jnp.uint32)
    tg = tab16.reshape(V, D // 32, 2, 16)
    packed = jax.lax.bitcast_convert_type(
        (tg[:, :, 0, :] | (tg[:, :, 1, :] << 16)).reshape(V, DP),
        jnp.int32)
    out = _prepare_encoder(src_flat, pos, packed)
    return out.reshape(src_word.shape)


# speedup vs baseline: 1.2332x; 1.2332x over previous
import functools

import jax
import jax.numpy as jnp
from jax import lax
from jax.experimental import pallas as pl
from jax.experimental.pallas import tpu as pltpu
from jax.experimental.pallas import tpu_sc as plsc

D = 768
DP = D // 2
V = 8192
B = 4 * 8192
NW = 32
RPW = B // NW
C = 8
NCHUNK = RPW // C
NB = 8
LANES = 16

_mesh = plsc.VectorSubcoreMesh(core_axis_name="c", subcore_axis_name="s")


@functools.partial(
    pl.kernel,
    mesh=_mesh,
    out_type=jax.ShapeDtypeStruct((B, D), jnp.float32),
    scratch_types=[
        pltpu.VMEM((NCHUNK, C), jnp.int32),
        pltpu.VMEM((NB, C, D), jnp.float32),
        pltpu.VMEM((NB, C, DP), jnp.int32),
    ] + [pltpu.SemaphoreType.DMA] * (3 * NB),
)
def _prepare_encoder(src_hbm, pos_hbm, tab_hbm, out_hbm,
                     idx_v, src_v, gath_v, *sems):
    sem_s = sems[0:NB]
    sem_g = sems[NB:2 * NB]
    sem_o = sems[2 * NB:3 * NB]
    wid = lax.axis_index("s") * 2 + lax.axis_index("c")
    base = wid * RPW
    pltpu.sync_copy(pos_hbm.at[wid], idx_v)

    def start_in(g, b):
        rb = base + g * C
        pltpu.async_copy(src_hbm.at[pl.ds(rb, C)], src_v.at[b], sem_s[b])
        pltpu.async_copy(tab_hbm.at[idx_v.at[g]], gath_v.at[b], sem_g[b])

    def wait_in(g, b):
        rb = base + g * C
        pltpu.make_async_copy(src_hbm.at[pl.ds(rb, C)], src_v.at[b],
                              sem_s[b]).wait()
        pltpu.make_async_copy(tab_hbm.at[idx_v.at[g]], gath_v.at[b],
                              sem_g[b]).wait()

    def start_out(g, b):
        rb = base + g * C
        pltpu.async_copy(src_v.at[b], out_hbm.at[pl.ds(rb, C)], sem_o[b])

    def wait_out(g, b):
        rb = base + g * C
        pltpu.make_async_copy(src_v.at[b], out_hbm.at[pl.ds(rb, C)],
                              sem_o[b]).wait()

    def compute(b):
        def row(r, cc):
            for c in range(DP // LANES):
                x = gath_v[b, r, pl.ds(c * LANES, LANES)]
                lo = jax.lax.bitcast_convert_type(x << 16, jnp.float32)
                hi = jax.lax.bitcast_convert_type(x & jnp.int32(-65536),
                                                  jnp.float32)
                plsc.addupdate(src_v.at[b, r, pl.ds(c * LANES, LANES)], lo)
                plsc.addupdate(src_v.at[b, r, pl.ds(DP + c * LANES, LANES)],
                               hi)
            return cc
        lax.fori_loop(0, C, row, 0)

    for g0 in range(NB):
        start_in(g0, g0)

    def block(i, carry):
        for bb in range(NB):
            g = NB * i + bb
            b2 = (bb + 4) % NB

            @pl.when(jnp.logical_and(g >= 4, g + 4 < NCHUNK))
            def _():
                wait_out(g - 4, b2)
                start_in(g + 4, b2)

            wait_in(g, bb)
            compute(bb)
            start_out(g, bb)
        return carry

    lax.fori_loop(0, NCHUNK // NB, block, 0)
    for k in range(NB):
        g = NCHUNK - NB + k
        wait_out(g, g % NB)


def kernel(src_word, src_pos, emb_table):
    src_flat = src_word.reshape(B, D)
    pos = src_pos.reshape(NW, NCHUNK, C)
    tab16 = jax.lax.bitcast_convert_type(
        emb_table.astype(jnp.bfloat16), jnp.uint16).astype(jnp.uint32)
    packed = jax.lax.bitcast_convert_type(
        tab16[:, :DP] | (tab16[:, DP:] << 16), jnp.int32)
    out = _prepare_encoder(src_flat, pos, packed)
    return out.reshape(src_word.shape)
